# baseline (speedup 1.0000x reference)
import jax
import jax.numpy as jnp
from jax import lax
from jax.experimental import pallas as pl
from jax.experimental.pallas import tpu as pltpu
from jax.experimental.pallas import tpu_sc as plsc

L = 16
NC = 2
NS = 16
NW = NC * NS

R = 128
N = 32768
RPW = R // NW
NCHUNK = N // L
GC = 4
NG = NCHUNK // GC
SLOT = N + 128
CAP = N // 2
NEG = -1e30


def _lane0(v):
  return lax.squeeze(lax.slice(v, (0,), (1,)), (0,))


def _vdiv_scalar(a, b):
  av = jnp.broadcast_to(a, (L,))
  bv = jnp.broadcast_to(b, (L,))
  return jnp.max(av / bv)


def _sc_body(x_hbm, b_hbm, out_hbm, buf, cval, cidx, gmaxb, hotg, bstage,
             sin0, sin1, sout0, sout1):
  wid = lax.axis_index("s") * NC + lax.axis_index("c")

  pltpu.sync_copy(b_hbm, bstage)
  bv = bstage[...]
  nbb = -1.0 / (1.0 + jnp.exp(-bv))

  iota = lax.iota(jnp.int32, L)

  def process_row(off):
    def p1_body(g, acc):
      gbase = off + g * GC * L
      gmax = buf[0, pl.ds(gbase, L)]
      for k in range(1, GC):
        gmax = jnp.maximum(gmax, buf[0, pl.ds(gbase + k * L, L)])
      red = gmax
      for sh in (8, 4, 2, 1):
        perm = jnp.bitwise_xor(iota, sh)
        shuf = lax.gather(
            red, perm[:, None],
            dimension_numbers=lax.GatherDimensionNumbers(
                offset_dims=(), collapsed_slice_dims=(0,),
                start_index_map=(0,)),
            slice_sizes=(1,),
            mode=lax.GatherScatterMode.PROMISE_IN_BOUNDS)
        red = jnp.maximum(red, shuf)
      gmaxb[pl.ds(g * L, L)] = red
      return jnp.maximum(acc, gmax)

    acc = lax.fori_loop(0, NG, p1_body,
                        jnp.full((L,), NEG, jnp.float32), unroll=4)
    m = jnp.max(acc)
    thr = m - 1.0

    def p2a_body(c, hcnt):
      gm = plsc.load_gather(gmaxb, [(iota + c * L) * L + (L - 1)])
      msk = gm > thr
      plsc.store_compressed(hotg.at[pl.ds(hcnt, L)], iota + c * L,
                            mask=msk)
      return hcnt + _lane0(plsc.all_reduce_population_count(msk))

    nhot = lax.fori_loop(0, NG // L, p2a_body, jnp.int32(0))

    def p2b_body(h, cnt):
      g = _lane0(hotg[pl.ds(h, L)])
      c = cnt
      for k in range(GC):
        cbase = (g * GC + k) * L
        v = buf[0, pl.ds(off + cbase, L)]
        msk = (v > thr) & (c < CAP - L)
        plsc.store_compressed(cval.at[pl.ds(c, L)], v, mask=msk)
        plsc.store_compressed(cidx.at[pl.ds(c, L)], iota + cbase,
                              mask=msk)
        c = c + _lane0(plsc.all_reduce_population_count(msk))
      return c

    ncand = lax.fori_loop(0, nhot, p2b_body, jnp.int32(0))
    cval[pl.ds(ncand, L)] = jnp.full((L,), NEG, jnp.float32)
    cidx[pl.ds(ncand, L)] = jnp.full((L,), N, jnp.int32)
    nch = lax.shift_right_logical(ncand + (L - 1), 4)

    def mich_cond(c):
      it, tau_prev, tau = c
      return (tau > tau_prev) & (it < 64)

    def mich_body(c):
      it, tau_prev, tau = c

      def sum_body(i, sk):
        s, k = sk
        v = cval[pl.ds(i * L, L)]
        msk = v > tau
        return (s + jnp.where(msk, v, 0.0),
                k + jnp.where(msk, 1.0, 0.0))

      s, k = lax.fori_loop(
          0, nch, sum_body,
          (jnp.zeros((L,), jnp.float32), jnp.zeros((L,), jnp.float32)))
      tau_new = _vdiv_scalar(jnp.sum(s) - 1.0, jnp.sum(k))
      return (it + 1, tau, tau_new)

    _, _, tau = lax.while_loop(mich_cond, mich_body,
                               (jnp.int32(0), m - 2.0, m - 1.0))

    def fill_body(i, c):
      buf[0, pl.ds(off + i * L, L)] = jnp.full((L,), 1.0, jnp.float32)
      return c

    lax.fori_loop(0, NCHUNK, fill_body, 0, unroll=16)

    def patch_body(i, c):
      v = cval[pl.ds(i * L, L)]
      o = jnp.exp(nbb * jnp.maximum(v - tau, 0.0))
      plsc.store_scatter(
          buf, [jnp.zeros((L,), jnp.int32), cidx[pl.ds(i * L, L)] + off], o)
      return c

    lax.fori_loop(0, nch, patch_body, 0)

  sins = (sin0, sin1)
  souts = (sout0, sout1)

  def copy_in(j):
    p = j & 1
    return pltpu.make_async_copy(x_hbm.at[pl.ds(wid * RPW + j, 1)],
                                 buf.at[:, pl.ds(p * SLOT, N)], sins[p])

  def copy_out(j):
    p = j & 1
    return pltpu.make_async_copy(buf.at[:, pl.ds(p * SLOT, N)],
                                 out_hbm.at[pl.ds(wid * RPW + j, 1)],
                                 souts[p])

  copy_in(0).start()
  for j in range(RPW):
    copy_in(j).wait()
    if j + 1 < RPW:
      if j >= 1:
        copy_out(j - 1).wait()
      copy_in(j + 1).start()
    process_row((j & 1) * SLOT)
    copy_out(j).start()
  copy_out(RPW - 2).wait()
  copy_out(RPW - 1).wait()


def kernel(x, b):
  bvec = jnp.full((L,), b, dtype=jnp.float32)
  mesh = plsc.VectorSubcoreMesh(core_axis_name="c", subcore_axis_name="s")
  out = pl.kernel(
      _sc_body,
      out_type=jax.ShapeDtypeStruct((R, N), jnp.float32),
      mesh=mesh,
      compiler_params=pltpu.CompilerParams(needs_layout_passes=False),
      scratch_types=[
          pltpu.VMEM((1, 2 * SLOT), jnp.float32),
          pltpu.VMEM((CAP + L,), jnp.float32),
          pltpu.VMEM((CAP + L,), jnp.int32),
          pltpu.VMEM((NG * L,), jnp.float32),
          pltpu.VMEM((NG + L,), jnp.int32),
          pltpu.VMEM((L,), jnp.float32),
          pltpu.SemaphoreType.DMA,
          pltpu.SemaphoreType.DMA,
          pltpu.SemaphoreType.DMA,
          pltpu.SemaphoreType.DMA,
      ],
  )(x, bvec)
  return out


# speedup vs baseline: 1.0868x; 1.0868x over previous
import jax
import jax.numpy as jnp
from jax import lax
from jax.experimental import pallas as pl
from jax.experimental.pallas import tpu as pltpu
from jax.experimental.pallas import tpu_sc as plsc

L = 16
NC = 2
NS = 16
NW = NC * NS

R = 128
N = 32768
RPW = R // NW
NCHUNK = N // L
GC = 4
NG = NCHUNK // GC
SLOT = N + 128
CAP = N // 2
NEG = -1e30


def _lane0(v):
  return lax.squeeze(lax.slice(v, (0,), (1,)), (0,))


def _vdiv_scalar(a, b):
  av = jnp.broadcast_to(a, (L,))
  bv = jnp.broadcast_to(b, (L,))
  return jnp.max(av / bv)


def _sc_body(x_hbm, b_hbm, out_hbm, buf, cval, cidx, gmaxb, hotg, bstage,
             sin0, sin1, sout0, sout1):
  wid = lax.axis_index("s") * NC + lax.axis_index("c")

  pltpu.sync_copy(b_hbm, bstage)
  bv = bstage[...]
  nbb = -1.0 / (1.0 + jnp.exp(-bv))

  iota = lax.iota(jnp.int32, L)

  def process_row(off):
    def p1_body(g, acc):
      gbase = off + g * GC * L
      gmax = buf[0, pl.ds(gbase, L)]
      for k in range(1, GC):
        gmax = jnp.maximum(gmax, buf[0, pl.ds(gbase + k * L, L)])
      gmaxb[pl.ds(g * L, L)] = plsc.cummax(gmax)
      return jnp.maximum(acc, gmax)

    acc = lax.fori_loop(0, NG, p1_body,
                        jnp.full((L,), NEG, jnp.float32), unroll=4)
    m = jnp.max(acc)
    thr = m - 1.0

    def p2a_body(c, hcnt):
      gm = plsc.load_gather(gmaxb, [(iota + c * L) * L + (L - 1)])
      msk = gm > thr
      plsc.store_compressed(hotg.at[pl.ds(hcnt, L)], iota + c * L,
                            mask=msk)
      return hcnt + _lane0(plsc.all_reduce_population_count(msk))

    nhot = lax.fori_loop(0, NG // L, p2a_body, jnp.int32(0))

    def p2b_body(h, cnt):
      g = _lane0(hotg[pl.ds(h, L)])
      c = cnt
      for k in range(GC):
        cbase = (g * GC + k) * L
        v = buf[0, pl.ds(off + cbase, L)]
        msk = (v > thr) & (c < CAP - L)
        plsc.store_compressed(cval.at[pl.ds(c, L)], v, mask=msk)
        plsc.store_compressed(cidx.at[pl.ds(c, L)], iota + cbase,
                              mask=msk)
        c = c + _lane0(plsc.all_reduce_population_count(msk))
      return c

    ncand = lax.fori_loop(0, nhot, p2b_body, jnp.int32(0))
    cval[pl.ds(ncand, L)] = jnp.full((L,), NEG, jnp.float32)
    cidx[pl.ds(ncand, L)] = jnp.full((L,), N, jnp.int32)
    nch = lax.shift_right_logical(ncand + (L - 1), 4)

    def mich_cond(c):
      it, tau_prev, tau = c
      return (tau > tau_prev) & (it < 64)

    def mich_body(c):
      it, tau_prev, tau = c

      def sum_body(i, sk):
        s, k = sk
        v = cval[pl.ds(i * L, L)]
        msk = v > tau
        return (s + jnp.where(msk, v, 0.0),
                k + jnp.where(msk, 1.0, 0.0))

      s, k = lax.fori_loop(
          0, nch, sum_body,
          (jnp.zeros((L,), jnp.float32), jnp.zeros((L,), jnp.float32)))
      tau_new = _vdiv_scalar(jnp.sum(s) - 1.0, jnp.sum(k))
      return (it + 1, tau, tau_new)

    _, _, tau = lax.while_loop(mich_cond, mich_body,
                               (jnp.int32(0), m - 2.0, m - 1.0))

    def fill_body(i, c):
      buf[0, pl.ds(off + i * L, L)] = jnp.full((L,), 1.0, jnp.float32)
      return c

    lax.fori_loop(0, NCHUNK, fill_body, 0, unroll=16)

    def patch_body(i, c):
      v = cval[pl.ds(i * L, L)]
      o = jnp.exp(nbb * jnp.maximum(v - tau, 0.0))
      plsc.store_scatter(
          buf, [jnp.zeros((L,), jnp.int32), cidx[pl.ds(i * L, L)] + off], o)
      return c

    lax.fori_loop(0, nch, patch_body, 0)

  sins = (sin0, sin1)
  souts = (sout0, sout1)

  def copy_in(j):
    p = j & 1
    return pltpu.make_async_copy(x_hbm.at[pl.ds(wid * RPW + j, 1)],
                                 buf.at[:, pl.ds(p * SLOT, N)], sins[p])

  def copy_out(j):
    p = j & 1
    return pltpu.make_async_copy(buf.at[:, pl.ds(p * SLOT, N)],
                                 out_hbm.at[pl.ds(wid * RPW + j, 1)],
                                 souts[p])

  copy_in(0).start()
  for j in range(RPW):
    copy_in(j).wait()
    if j + 1 < RPW:
      if j >= 1:
        copy_out(j - 1).wait()
      copy_in(j + 1).start()
    process_row((j & 1) * SLOT)
    copy_out(j).start()
  copy_out(RPW - 2).wait()
  copy_out(RPW - 1).wait()


def kernel(x, b):
  bvec = jnp.full((L,), b, dtype=jnp.float32)
  mesh = plsc.VectorSubcoreMesh(core_axis_name="c", subcore_axis_name="s")
  out = pl.kernel(
      _sc_body,
      out_type=jax.ShapeDtypeStruct((R, N), jnp.float32),
      mesh=mesh,
      compiler_params=pltpu.CompilerParams(needs_layout_passes=False),
      scratch_types=[
          pltpu.VMEM((1, 2 * SLOT), jnp.float32),
          pltpu.VMEM((CAP + L,), jnp.float32),
          pltpu.VMEM((CAP + L,), jnp.int32),
          pltpu.VMEM((NG * L,), jnp.float32),
          pltpu.VMEM((NG + L,), jnp.int32),
          pltpu.VMEM((L,), jnp.float32),
          pltpu.SemaphoreType.DMA,
          pltpu.SemaphoreType.DMA,
          pltpu.SemaphoreType.DMA,
          pltpu.SemaphoreType.DMA,
      ],
  )(x, bvec)
  return out
